# baseline (speedup 1.0000x reference)
import jax
import jax.numpy as jnp
from jax import lax
from jax.experimental import pallas as pl
from jax.experimental.pallas import tpu as pltpu
from jax.experimental.pallas import tpu_sc as plsc

B = 16384
L = 200
V = 128
D = 16
C = 4

NC = 2
NS = 16
NW = NC * NS
ROWS_PER_W = B // NW
GROUPS = ROWS_PER_W // 16

HI16 = -65536
RND = 0x8000


def _sc_body(ids_hbm, mask_hbm, et_hbm, wt_hbm, b_hbm, out_hbm,
             ids_v0, ids_v1, mask_v0, mask_v1, et_v, wt_v, b_v,
             t01, t23, out_v, s0, s1, sm0, sm1):
    wid = lax.axis_index("s") * NC + lax.axis_index("c")
    row0 = wid * ROWS_PER_W

    ibufs = (ids_v0, ids_v1)
    mbufs = (mask_v0, mask_v1)
    isems = (s0, s1)
    msems = (sm0, sm1)
    cps = {}

    def start(g):
        buf = g % 2
        r = row0 + g * 16
        cps[(g, "i")] = pltpu.async_copy(
            ids_hbm.at[pl.ds(r, 16), :], ibufs[buf], isems[buf])
        cps[(g, "m")] = pltpu.async_copy(
            mask_hbm.at[pl.ds(r, 16), :], mbufs[buf], msems[buf])

    start(0)

    pltpu.sync_copy(et_hbm, et_v)
    pltpu.sync_copy(wt_hbm, wt_v)
    pltpu.sync_copy(b_hbm, b_v)

    zf = jnp.zeros((16,), jnp.float32)
    zi = jnp.zeros((16,), jnp.int32)

    def rn_hi(x):
        return (plsc.bitcast(x, jnp.int32) + RND) & HI16

    for pair, (ca, cb) in enumerate(((0, 1), (2, 3))):
        tref = (t01, t23)[pair]
        wva = [
            plsc.load_gather(wt_v, [jnp.full((16,), 1 + ca * D + d, jnp.int32)])
            for d in range(D)
        ]
        wvb = [
            plsc.load_gather(wt_v, [jnp.full((16,), 1 + cb * D + d, jnp.int32)])
            for d in range(D)
        ]
        for k in range(V // 16):
            acca = zf
            accb = zf
            for d in range(D):
                ev = et_v[pl.ds(d * V + k * 16, 16)]
                acca = acca + ev * wva[d]
                accb = accb + ev * wvb[d]
            packed = rn_hi(acca) | lax.shift_right_logical(
                plsc.bitcast(accb, jnp.int32) + RND, 16)
            tref[pl.ds(k * 16, 16)] = packed
        for k in range(V // 16):
            tref[pl.ds(V + k * 16, 16)] = zi

    lane = lax.iota(jnp.int32, 16)
    bcasts = [
        plsc.load_gather(b_v, [jnp.full((16,), 1 + c, jnp.int32)])
        for c in range(C)
    ]

    for g in range(GROUPS):
        if g + 1 < GROUPS:
            start(g + 1)
        cps[(g, "i")].wait()
        cps[(g, "m")].wait()
        ids_v = ibufs[g % 2]
        mask_v = mbufs[g % 2]

        def body(l, carry):
            a0, a1, a2, a3, cnt = carry
            cols = jnp.full((16,), 0, jnp.int32) + l
            idv = plsc.load_gather(ids_v, [lane, cols])
            mkv = plsc.load_gather(mask_v, [lane, cols])
            eff = idv + (128 - (mkv << 7))
            g01 = plsc.load_gather(t01, [eff])
            g23 = plsc.load_gather(t23, [eff])
            a0 = a0 + plsc.bitcast(g01 & HI16, jnp.float32)
            a1 = a1 + plsc.bitcast(g01 << 16, jnp.float32)
            a2 = a2 + plsc.bitcast(g23 & HI16, jnp.float32)
            a3 = a3 + plsc.bitcast(g23 << 16, jnp.float32)
            cnt = cnt + mkv
            return a0, a1, a2, a3, cnt

        a0, a1, a2, a3, cnt = lax.fori_loop(
            0, L, body, (zf, zf, zf, zf, zi), unroll=4)

        den = jnp.maximum(cnt.astype(jnp.float32), 1.0)
        obase = (g * 16 + lane) * C
        for c, a in enumerate((a0, a1, a2, a3)):
            outv = a / den + bcasts[c]
            plsc.store_scatter(out_v, [obase + c], outv)

    pltpu.sync_copy(out_v, out_hbm.at[pl.ds(row0 * C, ROWS_PER_W * C)])


@jax.jit
def _run(input_ids, attention_mask, embedding_table, W, b):
    ids = input_ids.astype(jnp.int32)
    mask = attention_mask.astype(jnp.int32)
    et_flat = embedding_table.T.reshape(-1)
    wt_flat = jnp.pad(W.T.reshape(-1), (1, 15))
    b16 = jnp.pad(b.astype(jnp.float32), (1, 11))
    mesh = plsc.VectorSubcoreMesh(core_axis_name="c", subcore_axis_name="s",
                                  num_cores=NC, num_subcores=NS)
    f = pl.kernel(
        _sc_body,
        out_type=jax.ShapeDtypeStruct((B * C,), jnp.float32),
        mesh=mesh,
        compiler_params=pltpu.CompilerParams(needs_layout_passes=False),
        scratch_types=[
            pltpu.VMEM((16, L), jnp.int32),
            pltpu.VMEM((16, L), jnp.int32),
            pltpu.VMEM((16, L), jnp.int32),
            pltpu.VMEM((16, L), jnp.int32),
            pltpu.VMEM((D * V,), jnp.float32),
            pltpu.VMEM((80,), jnp.float32),
            pltpu.VMEM((16,), jnp.float32),
            pltpu.VMEM((2 * V,), jnp.int32),
            pltpu.VMEM((2 * V,), jnp.int32),
            pltpu.VMEM((ROWS_PER_W * C,), jnp.float32),
            pltpu.SemaphoreType.DMA,
            pltpu.SemaphoreType.DMA,
            pltpu.SemaphoreType.DMA,
            pltpu.SemaphoreType.DMA,
        ],
    )
    return f(ids, mask, et_flat, wt_flat, b16).reshape(B, C)


def kernel(input_ids, attention_mask, embedding_table, W, b):
    return _run(input_ids, attention_mask, embedding_table, W, b)


# speedup vs baseline: 1.6329x; 1.6329x over previous
import jax
import jax.numpy as jnp
from jax import lax
from jax.experimental import pallas as pl
from jax.experimental.pallas import tpu as pltpu
from jax.experimental.pallas import tpu_sc as plsc

B = 16384
L = 200
V = 128
D = 16
C = 4

NC = 2
NS = 16
NW = NC * NS
ROWS_PER_W = B // NW
GROUPS = ROWS_PER_W // 16


HI16 = -65536
RND = 0x8000


def _sc_body(eff_hbm, et_hbm, wt_hbm, b_hbm, out_hbm,
             eff_v0, eff_v1, et_v, wt_v, b_v,
             t01, t23, out_v, s0, s1):
    wid = lax.axis_index("s") * NC + lax.axis_index("c")
    row0 = wid * ROWS_PER_W

    ebufs = (eff_v0, eff_v1)
    sems = (s0, s1)
    cps = {}

    def start(g):
        buf = g % 2
        off = (row0 + g * 16) * L
        cps[g] = pltpu.async_copy(
            eff_hbm.at[pl.ds(off, 16 * L)], ebufs[buf], sems[buf])

    start(0)

    pltpu.sync_copy(et_hbm, et_v)
    pltpu.sync_copy(wt_hbm, wt_v)
    pltpu.sync_copy(b_hbm, b_v)

    zf = jnp.zeros((16,), jnp.float32)
    zi = jnp.zeros((16,), jnp.int32)

    def rn_hi(x):
        return (plsc.bitcast(x, jnp.int32) + RND) & HI16

    for pair, (ca, cb) in enumerate(((0, 1), (2, 3))):
        tref = (t01, t23)[pair]
        wva = [
            plsc.load_gather(wt_v, [jnp.full((16,), 1 + ca * D + d, jnp.int32)])
            for d in range(D)
        ]
        wvb = [
            plsc.load_gather(wt_v, [jnp.full((16,), 1 + cb * D + d, jnp.int32)])
            for d in range(D)
        ]
        for k in range(V // 16):
            acca = zf
            accb = zf
            for d in range(D):
                ev = et_v[pl.ds(d * V + k * 16, 16)]
                acca = acca + ev * wva[d]
                accb = accb + ev * wvb[d]
            packed = rn_hi(acca) | lax.shift_right_logical(
                plsc.bitcast(accb, jnp.int32) + RND, 16)
            tref[pl.ds(k * 16, 16)] = packed
        for k in range(V // 16):
            tref[pl.ds(V + k * 16, 16)] = zi

    lane = lax.iota(jnp.int32, 16)
    base_idx = lane * L
    bcasts = [
        plsc.load_gather(b_v, [jnp.full((16,), 1 + c, jnp.int32)])
        for c in range(C)
    ]
    lf = float(L)

    for g in range(GROUPS):
        if g + 1 < GROUPS:
            start(g + 1)
        cps[g].wait()
        eff_v = ebufs[g % 2]

        def body(l, carry):
            a0, a1, a2, a3, cout = carry
            eff = plsc.load_gather(eff_v, [base_idx + l])
            g01 = plsc.load_gather(t01, [eff])
            g23 = plsc.load_gather(t23, [eff])
            a0 = a0 + plsc.bitcast(g01 & HI16, jnp.float32)
            a1 = a1 + plsc.bitcast(g01 << 16, jnp.float32)
            a2 = a2 + plsc.bitcast(g23 & HI16, jnp.float32)
            a3 = a3 + plsc.bitcast(g23 << 16, jnp.float32)
            cout = cout + (eff >> 7)
            return a0, a1, a2, a3, cout

        a0, a1, a2, a3, cout = lax.fori_loop(
            0, L, body, (zf, zf, zf, zf, zi), unroll=8)

        den = jnp.maximum(lf - cout.astype(jnp.float32), 1.0)
        obase = (g * 16 + lane) * C
        for c, a in enumerate((a0, a1, a2, a3)):
            outv = a / den + bcasts[c]
            plsc.store_scatter(out_v, [obase + c], outv)

    pltpu.sync_copy(out_v, out_hbm.at[pl.ds(row0 * C, ROWS_PER_W * C)])


@jax.jit
def _run(input_ids, attention_mask, embedding_table, W, b):
    ids = input_ids.astype(jnp.int32)
    mask = attention_mask.astype(jnp.int32)
    eff_flat = (ids + ((1 - mask) << 7)).reshape(-1)
    et_flat = embedding_table.T.reshape(-1)
    wt_flat = jnp.pad(W.T.reshape(-1), (1, 15))
    b16 = jnp.pad(b.astype(jnp.float32), (1, 11))
    mesh = plsc.VectorSubcoreMesh(core_axis_name="c", subcore_axis_name="s",
                                  num_cores=NC, num_subcores=NS)
    f = pl.kernel(
        _sc_body,
        out_type=jax.ShapeDtypeStruct((B * C,), jnp.float32),
        mesh=mesh,
        compiler_params=pltpu.CompilerParams(needs_layout_passes=False),
        scratch_types=[
            pltpu.VMEM((16 * L,), jnp.int32),
            pltpu.VMEM((16 * L,), jnp.int32),
            pltpu.VMEM((D * V,), jnp.float32),
            pltpu.VMEM((80,), jnp.float32),
            pltpu.VMEM((16,), jnp.float32),
            pltpu.VMEM((2 * V,), jnp.int32),
            pltpu.VMEM((2 * V,), jnp.int32),
            pltpu.VMEM((ROWS_PER_W * C,), jnp.float32),
            pltpu.SemaphoreType.DMA,
            pltpu.SemaphoreType.DMA,
        ],
    )
    return f(eff_flat, et_flat, wt_flat, b16).reshape(B, C)


def kernel(input_ids, attention_mask, embedding_table, W, b):
    return _run(input_ids, attention_mask, embedding_table, W, b)


# speedup vs baseline: 1.6448x; 1.0073x over previous
import jax
import jax.numpy as jnp
from jax import lax
from jax.experimental import pallas as pl
from jax.experimental.pallas import tpu as pltpu
from jax.experimental.pallas import tpu_sc as plsc

B = 16384
L = 200
V = 128
D = 16
C = 4

NC = 2
NS = 16
NW = NC * NS
ROWS_PER_W = B // NW
GROUPS = ROWS_PER_W // 16


HI16 = -65536
RND = 0x8000


def _sc_body(eff_hbm, et_hbm, wt_hbm, b_hbm, out_hbm,
             eff_v0, eff_v1, et_v, wt_v, b_v,
             t01, t23, out_v, s0, s1):
    wid = lax.axis_index("s") * NC + lax.axis_index("c")
    row0 = wid * ROWS_PER_W

    ebufs = (eff_v0, eff_v1)
    sems = (s0, s1)
    cps = {}

    def start(g):
        buf = g % 2
        off = (row0 + g * 16) * L
        cps[g] = pltpu.async_copy(
            eff_hbm.at[pl.ds(off, 16 * L)], ebufs[buf], sems[buf])

    start(0)

    pltpu.sync_copy(et_hbm, et_v)
    pltpu.sync_copy(wt_hbm, wt_v)
    pltpu.sync_copy(b_hbm, b_v)

    zf = jnp.zeros((16,), jnp.float32)
    zi = jnp.zeros((16,), jnp.int32)

    def rn_hi(x):
        return (plsc.bitcast(x, jnp.int32) + RND) & HI16

    for pair, (ca, cb) in enumerate(((0, 1), (2, 3))):
        tref = (t01, t23)[pair]
        wva = [
            plsc.load_gather(wt_v, [jnp.full((16,), 1 + ca * D + d, jnp.int32)])
            for d in range(D)
        ]
        wvb = [
            plsc.load_gather(wt_v, [jnp.full((16,), 1 + cb * D + d, jnp.int32)])
            for d in range(D)
        ]
        for k in range(V // 16):
            acca = zf
            accb = zf
            for d in range(D):
                ev = et_v[pl.ds(d * V + k * 16, 16)]
                acca = acca + ev * wva[d]
                accb = accb + ev * wvb[d]
            packed = rn_hi(acca) | lax.shift_right_logical(
                plsc.bitcast(accb, jnp.int32) + RND, 16)
            tref[pl.ds(k * 16, 16)] = packed
        for k in range(V // 16):
            tref[pl.ds(V + k * 16, 16)] = zi

    lane = lax.iota(jnp.int32, 16)
    base_idx = lane * L
    bcasts = [
        plsc.load_gather(b_v, [jnp.full((16,), 1 + c, jnp.int32)])
        for c in range(C)
    ]
    lf = float(L)

    for g in range(GROUPS):
        if g + 1 < GROUPS:
            start(g + 1)
        cps[g].wait()
        eff_v = ebufs[g % 2]

        def body(l, carry):
            a0, a1, a2, a3, cout = carry
            eff = plsc.load_gather(eff_v, [base_idx + l])
            g01 = plsc.load_gather(t01, [eff])
            g23 = plsc.load_gather(t23, [eff])
            a0 = a0 + plsc.bitcast(g01 & HI16, jnp.float32)
            a1 = a1 + plsc.bitcast(g01 << 16, jnp.float32)
            a2 = a2 + plsc.bitcast(g23 & HI16, jnp.float32)
            a3 = a3 + plsc.bitcast(g23 << 16, jnp.float32)
            cout = cout + (eff >> 7)
            return a0, a1, a2, a3, cout

        a0, a1, a2, a3, cout = lax.fori_loop(
            0, L, body, (zf, zf, zf, zf, zi), unroll=4)

        den = jnp.maximum(lf - cout.astype(jnp.float32), 1.0)
        obase = (g * 16 + lane) * C
        for c, a in enumerate((a0, a1, a2, a3)):
            outv = a / den + bcasts[c]
            plsc.store_scatter(out_v, [obase + c], outv)

    pltpu.sync_copy(out_v, out_hbm.at[pl.ds(row0 * C, ROWS_PER_W * C)])


@jax.jit
def _run(input_ids, attention_mask, embedding_table, W, b):
    ids = input_ids.astype(jnp.int32)
    mask = attention_mask.astype(jnp.int32)
    eff_flat = (ids + ((1 - mask) << 7)).reshape(-1)
    et_flat = embedding_table.T.reshape(-1)
    wt_flat = jnp.pad(W.T.reshape(-1), (1, 15))
    b16 = jnp.pad(b.astype(jnp.float32), (1, 11))
    mesh = plsc.VectorSubcoreMesh(core_axis_name="c", subcore_axis_name="s",
                                  num_cores=NC, num_subcores=NS)
    f = pl.kernel(
        _sc_body,
        out_type=jax.ShapeDtypeStruct((B * C,), jnp.float32),
        mesh=mesh,
        compiler_params=pltpu.CompilerParams(needs_layout_passes=False),
        scratch_types=[
            pltpu.VMEM((16 * L,), jnp.int32),
            pltpu.VMEM((16 * L,), jnp.int32),
            pltpu.VMEM((D * V,), jnp.float32),
            pltpu.VMEM((80,), jnp.float32),
            pltpu.VMEM((16,), jnp.float32),
            pltpu.VMEM((2 * V,), jnp.int32),
            pltpu.VMEM((2 * V,), jnp.int32),
            pltpu.VMEM((ROWS_PER_W * C,), jnp.float32),
            pltpu.SemaphoreType.DMA,
            pltpu.SemaphoreType.DMA,
        ],
    )
    return f(eff_flat, et_flat, wt_flat, b16).reshape(B, C)


def kernel(input_ids, attention_mask, embedding_table, W, b):
    return _run(input_ids, attention_mask, embedding_table, W, b)
